# baseline (speedup 1.0000x reference)
import jax
import jax.numpy as jnp
from jax import lax
from jax.experimental import pallas as pl
from jax.experimental.pallas import tpu as pltpu
from jax.experimental.pallas import tpu_sc as plsc

B = 1024
SEQ = 77
D = 512
N_PRE = 7
N_SHARE = 16
N_ATTR = 15
N_SUF = 39
N_GATH = N_PRE + N_SUF
NC = 2
NS = 16
NW = NC * NS
BPW = B // NW
NBUF = 2
NGRP = N_GATH // NBUF


def _sc_body(gtab_hbm, vid_hbm, out_hbm, gtab_v, vid_v, buf0, buf1,
             sem0, sem1):
    bufs = [buf0, buf1]
    sems = [sem0, sem1]
    wid = lax.axis_index("s") * NC + lax.axis_index("c")
    base = wid * BPW
    pltpu.sync_copy(vid_hbm, vid_v)
    pltpu.sync_copy(gtab_hbm, gtab_v)
    c0 = vid_v[pl.ds(base, 16)]
    c1 = vid_v[pl.ds(base + 16, 16)]
    vids = [c0[i] for i in range(16)] + [c1[i] for i in range(16)]

    def group(g, _):
        for p in range(NBUF):
            s = g * NBUF + p
            s_out = jnp.where(s < N_PRE, s, s + (SEQ - N_GATH))
            dst = out_hbm.at[s_out, pl.ds(base, BPW)]

            @pl.when(g > 0)
            def _drain():
                pltpu.make_async_copy(bufs[p], dst, sems[p]).wait()

            for i in range(BPW):
                v = vids[i]
                for c in range(0, D, 16):
                    bufs[p][i, pl.ds(c, 16)] = gtab_v[v, s, pl.ds(c, 16)]
            pltpu.make_async_copy(bufs[p], dst, sems[p]).start()
        return ()

    lax.fori_loop(0, NGRP, group, (), unroll=False)
    for p in range(NBUF):
        pltpu.make_async_copy(
            bufs[p], out_hbm.at[0, pl.ds(base, BPW)], sems[p]).wait()


@jax.jit
def _sc_scatter(gtab, vid):
    mesh = plsc.VectorSubcoreMesh(core_axis_name="c", subcore_axis_name="s")
    f = pl.kernel(
        _sc_body,
        out_type=jax.ShapeDtypeStruct((SEQ, B, D), jnp.float32),
        mesh=mesh,
        scratch_types=[
            pltpu.VMEM((3, N_GATH, D), jnp.float32),
            pltpu.VMEM((B,), jnp.int32),
            pltpu.VMEM((BPW, D), jnp.float32),
            pltpu.VMEM((BPW, D), jnp.float32),
            pltpu.SemaphoreType.DMA,
            pltpu.SemaphoreType.DMA,
        ],
    )
    return f(gtab, vid)


def _tc_body(share_ref, attr_ref, out1_ref, out_ref, bbuf, sem_a, sem_b):
    h1 = pltpu.make_async_copy(
        attr_ref, out_ref.at[pl.ds(N_PRE + N_SHARE, N_ATTR)], sem_a)
    h1.start()
    bbuf[...] = jnp.broadcast_to(
        share_ref[...][:, None, :], (N_SHARE, B, D))
    h2 = pltpu.make_async_copy(
        bbuf, out_ref.at[pl.ds(N_PRE, N_SHARE)], sem_b)
    h2.start()
    h1.wait()
    h2.wait()


@jax.jit
def _tc_finish(out1, attr_t, share):
    return pl.pallas_call(
        _tc_body,
        out_shape=jax.ShapeDtypeStruct((SEQ, B, D), jnp.float32),
        in_specs=[
            pl.BlockSpec(memory_space=pltpu.VMEM),
            pl.BlockSpec(memory_space=pl.ANY),
            pl.BlockSpec(memory_space=pl.ANY),
        ],
        out_specs=pl.BlockSpec(memory_space=pl.ANY),
        scratch_shapes=[
            pltpu.VMEM((N_SHARE, B, D), jnp.float32),
            pltpu.SemaphoreType.DMA,
            pltpu.SemaphoreType.DMA,
        ],
        input_output_aliases={2: 0},
    )(share, attr_t, out1)


def kernel(attribute, viewids, token_prefix, token_suffix, share_vectors):
    gtab = jnp.concatenate([token_prefix[:, 0], token_suffix[:, 0]],
                           axis=1)
    attr_t = jnp.transpose(attribute, (1, 0, 2))
    vid = viewids.astype(jnp.int32)
    out1 = _sc_scatter(gtab, vid)
    out_t = _tc_finish(out1, attr_t, share_vectors)
    return jnp.transpose(out_t, (1, 0, 2))


# speedup vs baseline: 4.0400x; 4.0400x over previous
import jax
import jax.numpy as jnp
from jax import lax
from jax.experimental import pallas as pl
from jax.experimental.pallas import tpu as pltpu
from jax.experimental.pallas import tpu_sc as plsc

B = 1024
SEQ = 77
D = 512
N_PRE = 7
N_SHARE = 16
N_ATTR = 15
N_SUF = 39
N_GATH = N_PRE + N_SUF
NC = 2
NS = 16
NW = NC * NS
BPW = B // NW
NGRP = N_GATH // 2


def _row(s):
    return jnp.where(s < N_PRE, s, s + (SEQ - N_GATH))


def _sc_body(gtab_hbm, vid_hbm, out_hbm, vid_v, v46_v, idx0, idx1,
             buf0, buf1, sem_g0, sem_g1, sem_o0, sem_o1):
    idxs = [idx0, idx1]
    bufs = [buf0, buf1]
    sem_g = [sem_g0, sem_g1]
    sem_o = [sem_o0, sem_o1]
    wid = lax.axis_index("s") * NC + lax.axis_index("c")
    base = wid * BPW
    pltpu.sync_copy(vid_hbm, vid_v)
    for c in range(0, BPW, 16):
        v46_v[pl.ds(c, 16)] = vid_v[pl.ds(base + c, 16)] * N_GATH

    def gather(s, p):
        for c in range(0, BPW, 16):
            idxs[p][pl.ds(c, 16)] = v46_v[pl.ds(c, 16)] + s
        pltpu.make_async_copy(gtab_hbm.at[idxs[p]], bufs[p], sem_g[p]).start()

    def out_start(s, p):
        dst = out_hbm.at[_row(s), pl.ds(base, BPW)]
        pltpu.make_async_copy(bufs[p], dst, sem_o[p]).start()

    def gather_wait(p):
        pltpu.make_async_copy(gtab_hbm.at[idxs[p]], bufs[p], sem_g[p]).wait()

    def out_wait(p):
        dst = out_hbm.at[0, pl.ds(base, BPW)]
        pltpu.make_async_copy(bufs[p], dst, sem_o[p]).wait()

    gather(0, 0)

    def group(g, _):
        for j in range(2):
            s = g * 2 + j
            p = j
            pn = 1 - p

            @pl.when(s + 1 < N_GATH)
            def _launch_next():
                @pl.when(s >= 1)
                def _drain_out():
                    out_wait(pn)
                gather(s + 1, pn)

            gather_wait(p)
            out_start(s, p)
        return ()

    lax.fori_loop(0, NGRP, group, (), unroll=False)
    out_wait(0)
    out_wait(1)


@jax.jit
def _sc_scatter(gtab_flat, vid):
    mesh = plsc.VectorSubcoreMesh(core_axis_name="c", subcore_axis_name="s")
    f = pl.kernel(
        _sc_body,
        out_type=jax.ShapeDtypeStruct((SEQ, B, D), jnp.float32),
        mesh=mesh,
        scratch_types=[
            pltpu.VMEM((B,), jnp.int32),
            pltpu.VMEM((BPW,), jnp.int32),
            pltpu.VMEM((BPW,), jnp.int32),
            pltpu.VMEM((BPW,), jnp.int32),
            pltpu.VMEM((BPW, D), jnp.float32),
            pltpu.VMEM((BPW, D), jnp.float32),
            pltpu.SemaphoreType.DMA,
            pltpu.SemaphoreType.DMA,
            pltpu.SemaphoreType.DMA,
            pltpu.SemaphoreType.DMA,
        ],
    )
    return f(gtab_flat, vid)


def _tc_body(share_ref, attr_ref, out1_ref, out_ref):
    g = pl.program_id(0)

    @pl.when(g < N_SHARE)
    def _share():
        row = share_ref[pl.ds(g, 1), :]
        out_ref[...] = jnp.broadcast_to(row[:, None, :], (1, B, D))

    @pl.when(g >= N_SHARE)
    def _attr():
        out_ref[...] = attr_ref[...]


@jax.jit
def _tc_finish(out1, attr_t, share):
    return pl.pallas_call(
        _tc_body,
        grid=(N_SHARE + N_ATTR,),
        out_shape=jax.ShapeDtypeStruct((SEQ, B, D), jnp.float32),
        in_specs=[
            pl.BlockSpec((N_SHARE, D), lambda g: (0, 0)),
            pl.BlockSpec((1, B, D),
                         lambda g: (jnp.maximum(g - N_SHARE, 0), 0, 0)),
            pl.BlockSpec(memory_space=pl.ANY),
        ],
        out_specs=pl.BlockSpec((1, B, D), lambda g: (N_PRE + g, 0, 0)),
        input_output_aliases={2: 0},
    )(share, attr_t, out1)


def kernel(attribute, viewids, token_prefix, token_suffix, share_vectors):
    gtab = jnp.concatenate([token_prefix[:, 0], token_suffix[:, 0]],
                           axis=1)
    gtab_flat = gtab.reshape(3 * N_GATH, D)
    attr_t = jnp.transpose(attribute, (1, 0, 2))
    vid = viewids.astype(jnp.int32)
    out1 = _sc_scatter(gtab_flat, vid)
    out_t = _tc_finish(out1, attr_t, share_vectors)
    return jnp.transpose(out_t, (1, 0, 2))


# speedup vs baseline: 4.2496x; 1.0519x over previous
import jax
import jax.numpy as jnp
from jax import lax
from jax.experimental import pallas as pl
from jax.experimental.pallas import tpu as pltpu
from jax.experimental.pallas import tpu_sc as plsc

B = 1024
SEQ = 77
D = 512
N_PRE = 7
N_SHARE = 16
N_ATTR = 15
N_SUF = 39
N_GATH = N_PRE + N_SUF
NC = 2
NS = 16
NW = NC * NS
BPW = B // NW
NBUF = 6
LOOKAHEAD = 3


def _row(s):
    return jnp.where(s < N_PRE, s, s + (SEQ - N_GATH))


def _sc_body(gtab_hbm, vid_hbm, out_hbm, vid_v, v46_v,
             idx0, idx1, idx2, idx3, idx4, idx5,
             buf0, buf1, buf2, buf3, buf4, buf5,
             sem_g0, sem_g1, sem_g2, sem_g3, sem_g4, sem_g5,
             sem_o0, sem_o1, sem_o2, sem_o3, sem_o4, sem_o5):
    idxs = [idx0, idx1, idx2, idx3, idx4, idx5]
    bufs = [buf0, buf1, buf2, buf3, buf4, buf5]
    sem_g = [sem_g0, sem_g1, sem_g2, sem_g3, sem_g4, sem_g5]
    sem_o = [sem_o0, sem_o1, sem_o2, sem_o3, sem_o4, sem_o5]
    wid = lax.axis_index("s") * NC + lax.axis_index("c")
    base = wid * BPW
    pltpu.sync_copy(vid_hbm, vid_v)
    for c in range(0, BPW, 16):
        v46_v[pl.ds(c, 16)] = vid_v[pl.ds(base + c, 16)] * N_GATH

    def gather(s):
        p = s % NBUF
        for c in range(0, BPW, 16):
            idxs[p][pl.ds(c, 16)] = v46_v[pl.ds(c, 16)] + s
        pltpu.make_async_copy(gtab_hbm.at[idxs[p]], bufs[p], sem_g[p]).start()

    def out_start(s):
        p = s % NBUF
        dst = out_hbm.at[_row(s), pl.ds(base, BPW)]
        pltpu.make_async_copy(bufs[p], dst, sem_o[p]).start()

    def gather_wait(s):
        p = s % NBUF
        pltpu.make_async_copy(gtab_hbm.at[idxs[p]], bufs[p], sem_g[p]).wait()

    def out_wait(s):
        p = s % NBUF
        dst = out_hbm.at[0, pl.ds(base, BPW)]
        pltpu.make_async_copy(bufs[p], dst, sem_o[p]).wait()

    for s in range(LOOKAHEAD):
        gather(s)
    for s in range(N_GATH):
        if s + LOOKAHEAD < N_GATH:
            if s - (NBUF - LOOKAHEAD) >= 0:
                out_wait(s - (NBUF - LOOKAHEAD))
            gather(s + LOOKAHEAD)
        gather_wait(s)
        out_start(s)
    for s in range(N_GATH - 2 * (NBUF - LOOKAHEAD), N_GATH):
        out_wait(s)


@jax.jit
def _sc_scatter(gtab_flat, vid):
    mesh = plsc.VectorSubcoreMesh(core_axis_name="c", subcore_axis_name="s")
    f = pl.kernel(
        _sc_body,
        out_type=jax.ShapeDtypeStruct((SEQ, B, D), jnp.float32),
        mesh=mesh,
        scratch_types=(
            [pltpu.VMEM((B,), jnp.int32), pltpu.VMEM((BPW,), jnp.int32)]
            + [pltpu.VMEM((BPW,), jnp.int32) for _ in range(NBUF)]
            + [pltpu.VMEM((BPW, D), jnp.float32) for _ in range(NBUF)]
            + [pltpu.SemaphoreType.DMA for _ in range(2 * NBUF)]
        ),
    )
    return f(gtab_flat, vid)


def _tc_body(share_ref, attr_ref, out1_ref, out_ref):
    g = pl.program_id(0)

    @pl.when(g < N_SHARE)
    def _share():
        row = share_ref[pl.ds(g, 1), :]
        out_ref[...] = jnp.broadcast_to(row[:, None, :], (1, B, D))

    @pl.when(g >= N_SHARE)
    def _attr():
        out_ref[...] = attr_ref[...]


@jax.jit
def _tc_finish(out1, attr_t, share):
    return pl.pallas_call(
        _tc_body,
        grid=(N_SHARE + N_ATTR,),
        out_shape=jax.ShapeDtypeStruct((SEQ, B, D), jnp.float32),
        in_specs=[
            pl.BlockSpec((N_SHARE, D), lambda g: (0, 0)),
            pl.BlockSpec((1, B, D),
                         lambda g: (jnp.maximum(g - N_SHARE, 0), 0, 0)),
            pl.BlockSpec(memory_space=pl.ANY),
        ],
        out_specs=pl.BlockSpec((1, B, D), lambda g: (N_PRE + g, 0, 0)),
        input_output_aliases={2: 0},
    )(share, attr_t, out1)


def kernel(attribute, viewids, token_prefix, token_suffix, share_vectors):
    gtab = jnp.concatenate([token_prefix[:, 0], token_suffix[:, 0]],
                           axis=1)
    gtab_flat = gtab.reshape(3 * N_GATH, D)
    attr_t = jnp.transpose(attribute, (1, 0, 2))
    vid = viewids.astype(jnp.int32)
    out1 = _sc_scatter(gtab_flat, vid)
    out_t = _tc_finish(out1, attr_t, share_vectors)
    return jnp.transpose(out_t, (1, 0, 2))


# speedup vs baseline: 12.7658x; 3.0040x over previous
import jax
import jax.numpy as jnp
from jax import lax
from jax.experimental import pallas as pl
from jax.experimental.pallas import tpu as pltpu
from jax.experimental.pallas import tpu_sc as plsc

B = 1024
SEQ = 77
D = 512
N_PRE = 7
N_SHARE = 16
N_ATTR = 15
N_SUF = 39
N_GATH = N_PRE + N_SUF
SH_OFF = N_PRE
AT_OFF = N_PRE + N_SHARE
NC = 2
NS = 16
NW = NC * NS
BPW = B // NW
NBUF = 4
NTC = N_SHARE + N_GATH


def _sc_body(attr_hbm, out_hbm, buf0, buf1, buf2, buf3,
             sem_i0, sem_i1, sem_i2, sem_i3,
             sem_o0, sem_o1, sem_o2, sem_o3):
    bufs = [buf0, buf1, buf2, buf3]
    sem_i = [sem_i0, sem_i1, sem_i2, sem_i3]
    sem_o = [sem_o0, sem_o1, sem_o2, sem_o3]
    wid = lax.axis_index("s") * NC + lax.axis_index("c")
    base = wid * BPW

    def attr_in(k):
        p = k % NBUF
        return pltpu.async_copy(
            attr_hbm.at[k, pl.ds(base, BPW)], bufs[p], sem_i[p])

    h_in = [attr_in(k) for k in range(NBUF)]
    h_out = [None] * NBUF
    for k in range(N_ATTR):
        p = k % NBUF
        h_in[p].wait()
        h_out[p] = pltpu.async_copy(
            bufs[p], out_hbm.at[AT_OFF + k, pl.ds(base, BPW)], sem_o[p])
        if k + NBUF < N_ATTR:
            h_out[p].wait()
            h_out[p] = None
            h_in[p] = attr_in(k + NBUF)
    for h in h_out:
        if h is not None:
            h.wait()


@jax.jit
def _sc_patch(attr_t):
    mesh = plsc.VectorSubcoreMesh(core_axis_name="c", subcore_axis_name="s")
    f = pl.kernel(
        _sc_body,
        out_type=jax.ShapeDtypeStruct((SEQ, B, D), jnp.float32),
        mesh=mesh,
        scratch_types=(
            [pltpu.VMEM((BPW, D), jnp.float32) for _ in range(NBUF)]
            + [pltpu.SemaphoreType.DMA for _ in range(2 * NBUF)]
        ),
    )
    return f(attr_t)


def _tc_body(vid_ref, gtab_ref, share_ref, out1_ref, out_ref):
    g = pl.program_id(0)

    @pl.when(g < N_SHARE)
    def _share():
        row = share_ref[pl.ds(jnp.minimum(g, N_SHARE - 1), 1), :]
        out_ref[...] = jnp.broadcast_to(row[None], (1, B, D))

    @pl.when(g >= N_SHARE)
    def _gather():
        s = jnp.minimum(g - N_SHARE, N_GATH - 1)
        vb = vid_ref[...]
        t0 = jnp.broadcast_to(gtab_ref[pl.ds(0, 1), pl.ds(s, 1), :],
                              (1, B, D))
        t1 = jnp.broadcast_to(gtab_ref[pl.ds(1, 1), pl.ds(s, 1), :],
                              (1, B, D))
        t2 = jnp.broadcast_to(gtab_ref[pl.ds(2, 1), pl.ds(s, 1), :],
                              (1, B, D))
        out_ref[...] = jnp.where(vb == 0, t0, jnp.where(vb == 1, t1, t2))


def _out_row(g):
    s = g - N_SHARE
    return jnp.where(g < N_SHARE, N_PRE + g,
                     jnp.where(s < N_PRE, s, s + (SEQ - N_GATH)))


@jax.jit
def _tc_fill(out1, gtab, share, vid3):
    return pl.pallas_call(
        _tc_body,
        grid=(NTC,),
        out_shape=jax.ShapeDtypeStruct((SEQ, B, D), jnp.float32),
        in_specs=[
            pl.BlockSpec((1, B, 1), lambda g: (0, 0, 0)),
            pl.BlockSpec((3, N_GATH, D), lambda g: (0, 0, 0)),
            pl.BlockSpec((N_SHARE, D), lambda g: (0, 0)),
            pl.BlockSpec(memory_space=pl.ANY),
        ],
        out_specs=pl.BlockSpec((1, B, D), lambda g: (_out_row(g), 0, 0)),
        input_output_aliases={3: 0},
    )(vid3, gtab, share, out1)


def kernel(attribute, viewids, token_prefix, token_suffix, share_vectors):
    gtab = jnp.concatenate([token_prefix[:, 0], token_suffix[:, 0]],
                           axis=1)
    attr_t = jnp.transpose(attribute, (1, 0, 2))
    vid3 = viewids.astype(jnp.int32).reshape(1, B, 1)
    out1 = _sc_patch(attr_t)
    out_t = _tc_fill(out1, gtab, share_vectors, vid3)
    return jnp.transpose(out_t, (1, 0, 2))
